# baseline (speedup 1.0000x reference)
import jax
import jax.numpy as jnp
from jax import lax
from jax.experimental import pallas as pl
from jax.experimental.pallas import tpu as pltpu, tpu_sc as plsc

B = 4096
CTX = 77
D = 512
NH = 4
NM = 3
VSLOT = NH + 1
OSLOT = NH + 2 + NM

_info = plsc.get_sparse_core_info()
_NC = _info.num_cores
_NS = _info.num_subcores
NW = _NC * _NS
ROWS_PER_W = B // NW

G = 32
NG = B // G


def _sc_body(verb_hbm, obj_hbm, vidx_hbm, oidx_hbm, vout_hbm, oout_hbm,
             idx_v, rows_v, sem):
    wid = lax.axis_index("s") * _NC + lax.axis_index("c")
    off = wid * ROWS_PER_W
    pltpu.sync_copy(vidx_hbm.at[pl.ds(off, ROWS_PER_W)], idx_v)
    pltpu.async_copy(verb_hbm.at[idx_v], rows_v, sem).wait()
    pltpu.sync_copy(rows_v, vout_hbm.at[pl.ds(off, ROWS_PER_W)])
    pltpu.sync_copy(oidx_hbm.at[pl.ds(off, ROWS_PER_W)], idx_v)
    pltpu.async_copy(obj_hbm.at[idx_v], rows_v, sem).wait()
    pltpu.sync_copy(rows_v, oout_hbm.at[pl.ds(off, ROWS_PER_W)])


NBUF = 4


def _tc_body(tokid_sref, tokemb_hbm, pos_ref, ph_ref, pm_ref,
             vrows_ref, orows_ref, out_hbm, basebuf, *bufs_and_sems):
    bigs = bufs_and_sems[:NBUF]
    gsem = bufs_and_sems[NBUF]
    sems = bufs_and_sems[NBUF + 1:]
    g = pl.program_id(0)
    ng = pl.num_programs(0)

    @pl.when(g == 0)
    def _init():
        cps = []
        for r in range(CTX):
            cp = pltpu.make_async_copy(
                tokemb_hbm.at[pl.ds(tokid_sref[r], 1)],
                basebuf.at[pl.ds(r, 1)], gsem)
            cp.start()
            cps.append(cp)
        for cp in cps:
            cp.wait()
        basebuf[1:1 + NH, :] = ph_ref[...]
        basebuf[NH + 2:NH + 2 + NM, :] = pm_ref[...]
        base_val = basebuf[...] + pos_ref[...]
        for i in range(G):
            for big in bigs:
                big[i] = base_val

    vp = vrows_ref[...] + pos_ref[VSLOT, :][None, :]
    op = orows_ref[...] + pos_ref[OSLOT, :][None, :]

    def _handle(big, sem):
        @pl.when(g >= NBUF)
        def _wait_prev():
            pltpu.make_async_copy(
                big, out_hbm.at[pl.ds((g - NBUF) * G, G)], sem).wait()

        big[:, VSLOT, :] = vp
        big[:, OSLOT, :] = op
        pltpu.make_async_copy(big, out_hbm.at[pl.ds(g * G, G)], sem).start()

    for k in range(NBUF):
        @pl.when(g % NBUF == k)
        def _go(k=k):
            _handle(bigs[k], sems[k])

    @pl.when(g == ng - 1)
    def _drain():
        for k in range(NBUF):
            pltpu.make_async_copy(
                bigs[k], out_hbm.at[pl.ds(g * G, G)], sems[k]).wait()


def kernel(pair_idx, token_ids, token_embedding, positional_embedding,
           prompt_vectors_head, prompt_vectors_mid, verb_embedding,
           obj_embedding):
    vidx = pair_idx[:, 0].astype(jnp.int32)
    oidx = pair_idx[:, 1].astype(jnp.int32)
    tokid = token_ids.reshape(CTX).astype(jnp.int32)
    pos = positional_embedding.reshape(CTX, D)
    verb2d = verb_embedding.reshape(-1, D)
    obj2d = obj_embedding.reshape(-1, D)

    mesh = plsc.VectorSubcoreMesh(core_axis_name="c", subcore_axis_name="s")
    gather = pl.kernel(
        _sc_body,
        mesh=mesh,
        out_type=(
            jax.ShapeDtypeStruct((B, D), jnp.float32),
            jax.ShapeDtypeStruct((B, D), jnp.float32),
        ),
        scratch_types=[
            pltpu.VMEM((ROWS_PER_W,), jnp.int32),
            pltpu.VMEM((ROWS_PER_W, D), jnp.float32),
            pltpu.SemaphoreType.DMA,
        ],
    )
    vrows, orows = gather(verb2d, obj2d, vidx, oidx)

    assemble = pl.pallas_call(
        _tc_body,
        grid_spec=pltpu.PrefetchScalarGridSpec(
            num_scalar_prefetch=1,
            grid=(NG,),
            in_specs=[
                pl.BlockSpec(memory_space=pl.ANY),
                pl.BlockSpec((CTX, D), lambda g, s: (0, 0)),
                pl.BlockSpec((NH, D), lambda g, s: (0, 0)),
                pl.BlockSpec((NM, D), lambda g, s: (0, 0)),
                pl.BlockSpec((G, D), lambda g, s: (g, 0)),
                pl.BlockSpec((G, D), lambda g, s: (g, 0)),
            ],
            out_specs=pl.BlockSpec(memory_space=pl.ANY),
            scratch_shapes=(
                [pltpu.VMEM((CTX, D), jnp.float32)]
                + [pltpu.VMEM((G, CTX, D), jnp.float32)] * NBUF
                + [pltpu.SemaphoreType.DMA] * (NBUF + 1)
            ),
        ),
        out_shape=jax.ShapeDtypeStruct((B, CTX, D), jnp.float32),
    )
    return assemble(tokid, token_embedding, pos, prompt_vectors_head,
                    prompt_vectors_mid, vrows, orows)


# speedup vs baseline: 1.0075x; 1.0075x over previous
import jax
import jax.numpy as jnp
from jax import lax
from jax.experimental import pallas as pl
from jax.experimental.pallas import tpu as pltpu, tpu_sc as plsc

B = 4096
CTX = 77
D = 512
NH = 4
NM = 3
VSLOT = NH + 1
OSLOT = NH + 2 + NM

_info = plsc.get_sparse_core_info()
_NC = _info.num_cores
_NS = _info.num_subcores
NW = _NC * _NS
ROWS_PER_W = B // NW

G = 64
NG = B // G


def _sc_body(verb_hbm, obj_hbm, vidx_hbm, oidx_hbm, vout_hbm, oout_hbm,
             idx_v, rows_v, sem):
    wid = lax.axis_index("s") * _NC + lax.axis_index("c")
    off = wid * ROWS_PER_W
    pltpu.sync_copy(vidx_hbm.at[pl.ds(off, ROWS_PER_W)], idx_v)
    pltpu.async_copy(verb_hbm.at[idx_v], rows_v, sem).wait()
    pltpu.sync_copy(rows_v, vout_hbm.at[pl.ds(off, ROWS_PER_W)])
    pltpu.sync_copy(oidx_hbm.at[pl.ds(off, ROWS_PER_W)], idx_v)
    pltpu.async_copy(obj_hbm.at[idx_v], rows_v, sem).wait()
    pltpu.sync_copy(rows_v, oout_hbm.at[pl.ds(off, ROWS_PER_W)])


NBUF = 2


def _tc_body(tokid_sref, tokemb_hbm, pos_ref, ph_ref, pm_ref,
             vrows_ref, orows_ref, out_hbm, basebuf, *bufs_and_sems):
    bigs = bufs_and_sems[:NBUF]
    gsem = bufs_and_sems[NBUF]
    sems = bufs_and_sems[NBUF + 1:]
    g = pl.program_id(0)
    ng = pl.num_programs(0)

    @pl.when(g == 0)
    def _init():
        cps = []
        for r in range(CTX):
            cp = pltpu.make_async_copy(
                tokemb_hbm.at[pl.ds(tokid_sref[r], 1)],
                basebuf.at[pl.ds(r, 1)], gsem)
            cp.start()
            cps.append(cp)
        for cp in cps:
            cp.wait()
        basebuf[1:1 + NH, :] = ph_ref[...]
        basebuf[NH + 2:NH + 2 + NM, :] = pm_ref[...]
        base_val = basebuf[...] + pos_ref[...]
        for i in range(G):
            for big in bigs:
                big[i] = base_val

    vp = vrows_ref[...] + pos_ref[VSLOT, :][None, :]
    op = orows_ref[...] + pos_ref[OSLOT, :][None, :]

    def _handle(big, sem):
        @pl.when(g >= NBUF)
        def _wait_prev():
            pltpu.make_async_copy(
                big, out_hbm.at[pl.ds((g - NBUF) * G, G)], sem).wait()

        big[:, VSLOT, :] = vp
        big[:, OSLOT, :] = op
        pltpu.make_async_copy(big, out_hbm.at[pl.ds(g * G, G)], sem).start()

    for k in range(NBUF):
        @pl.when(g % NBUF == k)
        def _go(k=k):
            _handle(bigs[k], sems[k])

    @pl.when(g == ng - 1)
    def _drain():
        for k in range(NBUF):
            pltpu.make_async_copy(
                bigs[k], out_hbm.at[pl.ds(g * G, G)], sems[k]).wait()


def kernel(pair_idx, token_ids, token_embedding, positional_embedding,
           prompt_vectors_head, prompt_vectors_mid, verb_embedding,
           obj_embedding):
    vidx = pair_idx[:, 0].astype(jnp.int32)
    oidx = pair_idx[:, 1].astype(jnp.int32)
    tokid = token_ids.reshape(CTX).astype(jnp.int32)
    pos = positional_embedding.reshape(CTX, D)
    verb2d = verb_embedding.reshape(-1, D)
    obj2d = obj_embedding.reshape(-1, D)

    mesh = plsc.VectorSubcoreMesh(core_axis_name="c", subcore_axis_name="s")
    gather = pl.kernel(
        _sc_body,
        mesh=mesh,
        out_type=(
            jax.ShapeDtypeStruct((B, D), jnp.float32),
            jax.ShapeDtypeStruct((B, D), jnp.float32),
        ),
        scratch_types=[
            pltpu.VMEM((ROWS_PER_W,), jnp.int32),
            pltpu.VMEM((ROWS_PER_W, D), jnp.float32),
            pltpu.SemaphoreType.DMA,
        ],
    )
    vrows, orows = gather(verb2d, obj2d, vidx, oidx)

    assemble = pl.pallas_call(
        _tc_body,
        grid_spec=pltpu.PrefetchScalarGridSpec(
            num_scalar_prefetch=1,
            grid=(NG,),
            in_specs=[
                pl.BlockSpec(memory_space=pl.ANY),
                pl.BlockSpec((CTX, D), lambda g, s: (0, 0)),
                pl.BlockSpec((NH, D), lambda g, s: (0, 0)),
                pl.BlockSpec((NM, D), lambda g, s: (0, 0)),
                pl.BlockSpec((G, D), lambda g, s: (g, 0)),
                pl.BlockSpec((G, D), lambda g, s: (g, 0)),
            ],
            out_specs=pl.BlockSpec(memory_space=pl.ANY),
            scratch_shapes=(
                [pltpu.VMEM((CTX, D), jnp.float32)]
                + [pltpu.VMEM((G, CTX, D), jnp.float32)] * NBUF
                + [pltpu.SemaphoreType.DMA] * (NBUF + 1)
            ),
        ),
        out_shape=jax.ShapeDtypeStruct((B, CTX, D), jnp.float32),
    )
    return assemble(tokid, token_embedding, pos, prompt_vectors_head,
                    prompt_vectors_mid, vrows, orows)
